# baseline (speedup 1.0000x reference)
import jax
import jax.numpy as jnp
from jax.experimental import pallas as pl
from jax.experimental.pallas import tpu as pltpu

N = 8192
BM = 256
NBLK = N // BM
D = 128


def _fused_gnn(xp_ref, adj_ref, w1_ref, w2_ref, w3_ref, wd1_ref, wd2_ref,
               wd3_ref, bias_ref, out_ref, ya, yb, g_acc):
    l = pl.program_id(0)
    i = pl.program_id(1)

    @pl.when(jnp.logical_and(l == 0, i == 0))
    def _init():
        ya[...] = jnp.dot(xp_ref[...], w1_ref[...],
                          preferred_element_type=jnp.float32)
        g_acc[...] = jnp.zeros_like(g_acc)

    a_blk = adj_ref[...]

    @pl.when(l == 0)
    def _layer0():
        h = jnp.maximum(
            jnp.dot(a_blk, ya[...], preferred_element_type=jnp.float32,
                    precision=jax.lax.Precision.HIGHEST)
            + bias_ref[0, :][None, :], 0.0)
        yb[pl.ds(i * BM, BM), :] = jnp.dot(
            h, w2_ref[...], preferred_element_type=jnp.float32)

    @pl.when(l == 1)
    def _layer1():
        h = jnp.maximum(
            jnp.dot(a_blk, yb[...], preferred_element_type=jnp.float32,
                    precision=jax.lax.Precision.HIGHEST)
            + bias_ref[1, :][None, :], 0.0)
        ya[pl.ds(i * BM, BM), :] = jnp.dot(
            h, w3_ref[...], preferred_element_type=jnp.float32)

    @pl.when(l == 2)
    def _layer2():
        h = jnp.maximum(
            jnp.dot(a_blk, ya[...], preferred_element_type=jnp.float32,
                    precision=jax.lax.Precision.HIGHEST)
            + bias_ref[2, :][None, :], 0.0)
        g_acc[...] += jnp.sum(h, axis=0, keepdims=True)

    @pl.when(jnp.logical_and(l == 2, i == NBLK - 1))
    def _head():
        g = g_acc[...]
        norm = jnp.maximum(jnp.sqrt(jnp.sum(g * g)), 1e-12)
        gn = g / norm
        d1 = jnp.maximum(
            jnp.dot(gn, wd1_ref[...], preferred_element_type=jnp.float32)
            + bias_ref[3, :][None, :], 0.0)
        d2 = jnp.maximum(
            jnp.dot(d1, wd2_ref[...], preferred_element_type=jnp.float32)
            + bias_ref[4, :][None, :], 0.0)
        d3 = (jnp.dot(d2, wd3_ref[...], preferred_element_type=jnp.float32)
              + bias_ref[5, :][None, :])
        out_ref[...] = jnp.broadcast_to(d3, out_ref.shape)


def _pad2(w, rows, cols):
    return jnp.pad(w, ((0, rows - w.shape[0]), (0, cols - w.shape[1])))


def kernel(x, adj, W1, b1, W2, b2, W3, b3, Wd1, bd1, Wd2, bd2, Wd3, bd3):
    xp = jnp.pad(x, ((0, 0), (0, D - x.shape[1])))
    w1 = _pad2(W1, D, D)
    w2 = _pad2(W2, D, D)
    w3 = _pad2(W3, D, D)
    wd1 = _pad2(Wd1, D, D)
    wd2 = _pad2(Wd2, D, D)
    wd3 = _pad2(Wd3, D, D)
    bias = jnp.zeros((8, D), jnp.float32)
    bias = bias.at[0, :16].set(b1).at[1, :32].set(b2).at[2, :64].set(b3)
    bias = bias.at[3, :128].set(bd1).at[4, :64].set(bd2).at[5, :1].set(bd3)

    full = lambda shape: pl.BlockSpec(shape, lambda l, i: (0,) * len(shape))
    out = pl.pallas_call(
        _fused_gnn,
        grid=(3, NBLK),
        in_specs=[
            full((N, D)),
            pl.BlockSpec((BM, N), lambda l, i: (i, 0)),
            full((D, D)), full((D, D)), full((D, D)),
            full((D, D)), full((D, D)), full((D, D)),
            full((8, D)),
        ],
        out_specs=pl.BlockSpec((8, D), lambda l, i: (0, 0)),
        out_shape=jax.ShapeDtypeStruct((8, D), jnp.float32),
        scratch_shapes=[
            pltpu.VMEM((N, D), jnp.float32),
            pltpu.VMEM((N, D), jnp.float32),
            pltpu.VMEM((1, D), jnp.float32),
        ],
        compiler_params=pltpu.CompilerParams(
            dimension_semantics=("arbitrary", "arbitrary")),
    )(xp, adj, w1, w2, w3, wd1, wd2, wd3, bias)
    return out[0, 0:1]


# speedup vs baseline: 2.6074x; 2.6074x over previous
import jax
import jax.numpy as jnp
from jax.experimental import pallas as pl
from jax.experimental.pallas import tpu as pltpu

N = 8192
BM = 256
NBLK = N // BM
D = 128


def _fused_gnn(xp_ref, adj_ref, w1_ref, w2_ref, w3_ref, wd1_ref, wd2_ref,
               wd3_ref, bias_ref, out_ref, ya, yb, g_acc):
    l = pl.program_id(0)
    i = pl.program_id(1)

    @pl.when(jnp.logical_and(l == 0, i == 0))
    def _init():
        ya[...] = jnp.dot(xp_ref[...], w1_ref[...],
                          preferred_element_type=jnp.float32)
        g_acc[...] = jnp.zeros_like(g_acc)

    a_blk = adj_ref[...]

    @pl.when(l == 0)
    def _layer0():
        h = jnp.maximum(
            jnp.dot(a_blk, ya[...], preferred_element_type=jnp.float32)
            + bias_ref[0, :][None, :], 0.0)
        yb[pl.ds(i * BM, BM), :] = jnp.dot(
            h, w2_ref[...], preferred_element_type=jnp.float32)

    @pl.when(l == 1)
    def _layer1():
        h = jnp.maximum(
            jnp.dot(a_blk, yb[...], preferred_element_type=jnp.float32)
            + bias_ref[1, :][None, :], 0.0)
        ya[pl.ds(i * BM, BM), :] = jnp.dot(
            h, w3_ref[...], preferred_element_type=jnp.float32)

    @pl.when(l == 2)
    def _layer2():
        h = jnp.maximum(
            jnp.dot(a_blk, ya[...], preferred_element_type=jnp.float32)
            + bias_ref[2, :][None, :], 0.0)
        g_acc[...] += jnp.sum(h, axis=0, keepdims=True)

    @pl.when(jnp.logical_and(l == 2, i == NBLK - 1))
    def _head():
        g = g_acc[...]
        norm = jnp.maximum(jnp.sqrt(jnp.sum(g * g)), 1e-12)
        gn = g / norm
        d1 = jnp.maximum(
            jnp.dot(gn, wd1_ref[...], preferred_element_type=jnp.float32)
            + bias_ref[3, :][None, :], 0.0)
        d2 = jnp.maximum(
            jnp.dot(d1, wd2_ref[...], preferred_element_type=jnp.float32)
            + bias_ref[4, :][None, :], 0.0)
        d3 = (jnp.dot(d2, wd3_ref[...], preferred_element_type=jnp.float32)
              + bias_ref[5, :][None, :])
        out_ref[...] = jnp.broadcast_to(d3, out_ref.shape)


def _pad2(w, rows, cols):
    return jnp.pad(w, ((0, rows - w.shape[0]), (0, cols - w.shape[1])))


def kernel(x, adj, W1, b1, W2, b2, W3, b3, Wd1, bd1, Wd2, bd2, Wd3, bd3):
    xp = jnp.pad(x, ((0, 0), (0, D - x.shape[1])))
    w1 = _pad2(W1, D, D)
    w2 = _pad2(W2, D, D)
    w3 = _pad2(W3, D, D)
    wd1 = _pad2(Wd1, D, D)
    wd2 = _pad2(Wd2, D, D)
    wd3 = _pad2(Wd3, D, D)
    bias = jnp.zeros((8, D), jnp.float32)
    bias = bias.at[0, :16].set(b1).at[1, :32].set(b2).at[2, :64].set(b3)
    bias = bias.at[3, :128].set(bd1).at[4, :64].set(bd2).at[5, :1].set(bd3)

    full = lambda shape: pl.BlockSpec(shape, lambda l, i: (0,) * len(shape))
    out = pl.pallas_call(
        _fused_gnn,
        grid=(3, NBLK),
        in_specs=[
            full((N, D)),
            pl.BlockSpec((BM, N), lambda l, i: (i, 0)),
            full((D, D)), full((D, D)), full((D, D)),
            full((D, D)), full((D, D)), full((D, D)),
            full((8, D)),
        ],
        out_specs=pl.BlockSpec((8, D), lambda l, i: (0, 0)),
        out_shape=jax.ShapeDtypeStruct((8, D), jnp.float32),
        scratch_shapes=[
            pltpu.VMEM((N, D), jnp.float32),
            pltpu.VMEM((N, D), jnp.float32),
            pltpu.VMEM((1, D), jnp.float32),
        ],
        compiler_params=pltpu.CompilerParams(
            dimension_semantics=("arbitrary", "arbitrary")),
    )(xp, adj, w1, w2, w3, wd1, wd2, wd3, bias)
    return out[0, 0:1]


# speedup vs baseline: 2.9404x; 1.1277x over previous
import jax
import jax.numpy as jnp
from jax.experimental import pallas as pl
from jax.experimental.pallas import tpu as pltpu

N = 8192
BMA = 256
NBA = N // BMA
BMB = 512
NBB = N // BMB
D = 128
HIGHEST = jax.lax.Precision.HIGHEST


def _pack_hi_lo(y):
    hi = y.astype(jnp.bfloat16)
    lo = (y - hi.astype(jnp.float32)).astype(jnp.bfloat16)
    return jnp.concatenate([hi[:, :64], lo[:, :64]], axis=1)


def _layer1_kernel(xp_ref, adj_ref, w1_ref, w2_ref, b1_ref, u_ref, y2_ref,
                   y1p, cs, y1f):
    i = pl.program_id(0)

    @pl.when(i == 0)
    def _init():
        y1f[...] = jnp.dot(xp_ref[...], w1_ref[...],
                           preferred_element_type=jnp.float32,
                           precision=HIGHEST)
        y1p[...] = _pack_hi_lo(y1f[...])
        cs[...] = 0.5 * jnp.sum(y1f[...], axis=0, keepdims=True)

    u = (adj_ref[...] - 0.5).astype(jnp.bfloat16)
    u_ref[...] = u
    c = jnp.dot(u, y1p[...], preferred_element_type=jnp.float32)
    h1 = jnp.maximum(
        c[:, :64] + c[:, 64:] + cs[0:1, :64] + b1_ref[0:1, :64], 0.0)
    y2_ref[...] = jnp.dot(h1, w2_ref[...],
                          preferred_element_type=jnp.float32,
                          precision=HIGHEST)


def _layer23_kernel(u_ref, y2_ref, w3_ref, wd1_ref, wd2_ref, wd3_ref, b_ref,
                    out_ref, ypk, cs, ynx, g):
    l = pl.program_id(0)
    i = pl.program_id(1)

    @pl.when(jnp.logical_and(l == 0, i == 0))
    def _start2():
        ypk[...] = _pack_hi_lo(y2_ref[...])
        cs[...] = 0.5 * jnp.sum(y2_ref[...], axis=0, keepdims=True)
        g[...] = jnp.zeros_like(g)

    @pl.when(jnp.logical_and(l == 1, i == 0))
    def _start3():
        ypk[...] = _pack_hi_lo(ynx[...])
        cs[...] = 0.5 * jnp.sum(ynx[...], axis=0, keepdims=True)

    c = jnp.dot(u_ref[...], ypk[...], preferred_element_type=jnp.float32)
    hsum = c[:, :64] + c[:, 64:] + cs[0:1, :64]

    @pl.when(l == 0)
    def _layer2():
        h2 = jnp.maximum(hsum + b_ref[1:2, :64], 0.0)
        ynx[pl.ds(i * BMB, BMB), :] = jnp.dot(
            h2, w3_ref[...], preferred_element_type=jnp.float32,
            precision=HIGHEST)

    @pl.when(l == 1)
    def _layer3():
        h3 = jnp.maximum(hsum + b_ref[2:3, :64], 0.0)
        g[...] += jnp.sum(h3, axis=0, keepdims=True)

    @pl.when(jnp.logical_and(l == 1, i == NBB - 1))
    def _head():
        gv = g[...]
        norm = jnp.maximum(jnp.sqrt(jnp.sum(gv * gv)), 1e-12)
        gn = gv / norm
        d1 = jnp.maximum(
            jnp.dot(gn, wd1_ref[...], preferred_element_type=jnp.float32,
                    precision=HIGHEST) + b_ref[3:4, :], 0.0)
        d2 = jnp.maximum(
            jnp.dot(d1, wd2_ref[...], preferred_element_type=jnp.float32,
                    precision=HIGHEST) + b_ref[4:5, :64], 0.0)
        d3 = (jnp.dot(d2, wd3_ref[...], preferred_element_type=jnp.float32,
                      precision=HIGHEST) + b_ref[5:6, :])
        out_ref[...] = jnp.broadcast_to(d3, out_ref.shape)


def _pad2(w, rows, cols):
    return jnp.pad(w, ((0, rows - w.shape[0]), (0, cols - w.shape[1])))


def kernel(x, adj, W1, b1, W2, b2, W3, b3, Wd1, bd1, Wd2, bd2, Wd3, bd3):
    xp = jnp.pad(x, ((0, 0), (0, D - x.shape[1])))
    w1 = _pad2(W1, D, D)
    w2 = _pad2(W2, 64, D)
    w3 = _pad2(W3, 64, D)
    wd1 = _pad2(Wd1, 64, D)
    wd2 = _pad2(Wd2, D, 64)
    wd3 = _pad2(Wd3, 64, D)
    bias = jnp.zeros((8, D), jnp.float32)
    bias = bias.at[0, :16].set(b1).at[1, :32].set(b2).at[2, :64].set(b3)
    bias = bias.at[3, :128].set(bd1).at[4, :64].set(bd2).at[5, :1].set(bd3)

    u, y2 = pl.pallas_call(
        _layer1_kernel,
        grid=(NBA,),
        in_specs=[
            pl.BlockSpec((N, D), lambda i: (0, 0)),
            pl.BlockSpec((BMA, N), lambda i: (i, 0)),
            pl.BlockSpec((D, D), lambda i: (0, 0)),
            pl.BlockSpec((64, D), lambda i: (0, 0)),
            pl.BlockSpec((8, D), lambda i: (0, 0)),
        ],
        out_specs=[
            pl.BlockSpec((BMA, N), lambda i: (i, 0)),
            pl.BlockSpec((BMA, D), lambda i: (i, 0)),
        ],
        out_shape=[
            jax.ShapeDtypeStruct((N, N), jnp.bfloat16),
            jax.ShapeDtypeStruct((N, D), jnp.float32),
        ],
        scratch_shapes=[
            pltpu.VMEM((N, D), jnp.bfloat16),
            pltpu.VMEM((1, D), jnp.float32),
            pltpu.VMEM((N, D), jnp.float32),
        ],
        compiler_params=pltpu.CompilerParams(
            dimension_semantics=("arbitrary",)),
    )(xp, adj, w1, w2, bias)

    out = pl.pallas_call(
        _layer23_kernel,
        grid=(2, NBB),
        in_specs=[
            pl.BlockSpec((BMB, N), lambda l, i: (i, 0)),
            pl.BlockSpec((N, D), lambda l, i: (0, 0)),
            pl.BlockSpec((64, D), lambda l, i: (0, 0)),
            pl.BlockSpec((64, D), lambda l, i: (0, 0)),
            pl.BlockSpec((D, 64), lambda l, i: (0, 0)),
            pl.BlockSpec((64, D), lambda l, i: (0, 0)),
            pl.BlockSpec((8, D), lambda l, i: (0, 0)),
        ],
        out_specs=pl.BlockSpec((8, D), lambda l, i: (0, 0)),
        out_shape=jax.ShapeDtypeStruct((8, D), jnp.float32),
        scratch_shapes=[
            pltpu.VMEM((N, D), jnp.bfloat16),
            pltpu.VMEM((1, D), jnp.float32),
            pltpu.VMEM((N, D), jnp.float32),
            pltpu.VMEM((1, 64), jnp.float32),
        ],
        compiler_params=pltpu.CompilerParams(
            dimension_semantics=("arbitrary", "arbitrary")),
    )(u, y2, w3, wd1, wd2, wd3, bias)
    return out[0, 0:1]


# speedup vs baseline: 3.0563x; 1.0394x over previous
import jax
import jax.numpy as jnp
from jax.experimental import pallas as pl
from jax.experimental.pallas import tpu as pltpu

N = 8192
BMA = 256
NBA = N // BMA
BMB = 1024
NBB = N // BMB
D = 128
HIGHEST = jax.lax.Precision.HIGHEST


def _pack_hi_lo(y):
    bits = jax.lax.bitcast_convert_type(y, jnp.uint32)
    hi = jax.lax.bitcast_convert_type(
        bits & jnp.uint32(0xFFFF0000), jnp.float32)
    lo = (y - hi).astype(jnp.bfloat16)
    return jnp.concatenate(
        [hi.astype(jnp.bfloat16)[:, :64], lo[:, :64]], axis=1)


def _layer1_kernel(xp_ref, adj_ref, w1_ref, w2_ref, b1_ref, u_ref, y2_ref,
                   y1p, cs, y1f):
    i = pl.program_id(0)

    @pl.when(i == 0)
    def _init():
        y1f[...] = jnp.dot(xp_ref[...], w1_ref[...],
                           preferred_element_type=jnp.float32,
                           precision=HIGHEST)
        y1p[...] = _pack_hi_lo(y1f[...])
        cs[...] = 0.5 * jnp.sum(y1f[...], axis=0, keepdims=True)

    u = (adj_ref[...] - 0.5).astype(jnp.bfloat16)
    u_ref[...] = u
    c = jnp.dot(u, y1p[...], preferred_element_type=jnp.float32)
    h1 = jnp.maximum(
        c[:, :64] + c[:, 64:] + cs[0:1, :64] + b1_ref[0:1, :64], 0.0)
    y2_ref[...] = jnp.dot(h1, w2_ref[...],
                          preferred_element_type=jnp.float32,
                          precision=HIGHEST)


def _layer23_kernel(u_ref, y2_ref, w3_ref, wd1_ref, wd2_ref, wd3_ref, b_ref,
                    out_ref, ypk, cs, ynx, g):
    l = pl.program_id(0)
    i = pl.program_id(1)

    @pl.when(jnp.logical_and(l == 0, i == 0))
    def _start2():
        ypk[...] = _pack_hi_lo(y2_ref[...])
        cs[...] = 0.5 * jnp.sum(y2_ref[...], axis=0, keepdims=True)
        g[...] = jnp.zeros_like(g)

    @pl.when(jnp.logical_and(l == 1, i == 0))
    def _start3():
        ypk[...] = _pack_hi_lo(ynx[...])
        cs[...] = 0.5 * jnp.sum(ynx[...], axis=0, keepdims=True)

    c = jnp.dot(u_ref[...], ypk[...], preferred_element_type=jnp.float32)
    hsum = c[:, :64] + c[:, 64:] + cs[0:1, :64]

    @pl.when(l == 0)
    def _layer2():
        h2 = jnp.maximum(hsum + b_ref[1:2, :64], 0.0)
        ynx[pl.ds(i * BMB, BMB), :] = jnp.dot(
            h2, w3_ref[...], preferred_element_type=jnp.float32,
            precision=HIGHEST)

    @pl.when(l == 1)
    def _layer3():
        h3 = jnp.maximum(hsum + b_ref[2:3, :64], 0.0)
        g[...] += jnp.sum(h3, axis=0, keepdims=True)

    @pl.when(jnp.logical_and(l == 1, i == NBB - 1))
    def _head():
        gv = g[...]
        norm = jnp.maximum(jnp.sqrt(jnp.sum(gv * gv)), 1e-12)
        gn = gv / norm
        d1 = jnp.maximum(
            jnp.dot(gn, wd1_ref[...], preferred_element_type=jnp.float32,
                    precision=HIGHEST) + b_ref[3:4, :], 0.0)
        d2 = jnp.maximum(
            jnp.dot(d1, wd2_ref[...], preferred_element_type=jnp.float32,
                    precision=HIGHEST) + b_ref[4:5, :64], 0.0)
        d3 = (jnp.dot(d2, wd3_ref[...], preferred_element_type=jnp.float32,
                      precision=HIGHEST) + b_ref[5:6, :])
        out_ref[...] = jnp.broadcast_to(d3, out_ref.shape)


def _pad2(w, rows, cols):
    return jnp.pad(w, ((0, rows - w.shape[0]), (0, cols - w.shape[1])))


def kernel(x, adj, W1, b1, W2, b2, W3, b3, Wd1, bd1, Wd2, bd2, Wd3, bd3):
    xp = jnp.pad(x, ((0, 0), (0, D - x.shape[1])))
    w1 = _pad2(W1, D, D)
    w2 = _pad2(W2, 64, D)
    w3 = _pad2(W3, 64, D)
    wd1 = _pad2(Wd1, 64, D)
    wd2 = _pad2(Wd2, D, 64)
    wd3 = _pad2(Wd3, 64, D)
    bias = jnp.zeros((8, D), jnp.float32)
    bias = bias.at[0, :16].set(b1).at[1, :32].set(b2).at[2, :64].set(b3)
    bias = bias.at[3, :128].set(bd1).at[4, :64].set(bd2).at[5, :1].set(bd3)

    u, y2 = pl.pallas_call(
        _layer1_kernel,
        grid=(NBA,),
        in_specs=[
            pl.BlockSpec((N, D), lambda i: (0, 0)),
            pl.BlockSpec((BMA, N), lambda i: (i, 0)),
            pl.BlockSpec((D, D), lambda i: (0, 0)),
            pl.BlockSpec((64, D), lambda i: (0, 0)),
            pl.BlockSpec((8, D), lambda i: (0, 0)),
        ],
        out_specs=[
            pl.BlockSpec((BMA, N), lambda i: (i, 0)),
            pl.BlockSpec((BMA, D), lambda i: (i, 0)),
        ],
        out_shape=[
            jax.ShapeDtypeStruct((N, N), jnp.bfloat16),
            jax.ShapeDtypeStruct((N, D), jnp.float32),
        ],
        scratch_shapes=[
            pltpu.VMEM((N, D), jnp.bfloat16),
            pltpu.VMEM((1, D), jnp.float32),
            pltpu.VMEM((N, D), jnp.float32),
        ],
        compiler_params=pltpu.CompilerParams(
            dimension_semantics=("arbitrary",)),
    )(xp, adj, w1, w2, bias)

    out = pl.pallas_call(
        _layer23_kernel,
        grid=(2, NBB),
        in_specs=[
            pl.BlockSpec((BMB, N), lambda l, i: (i, 0)),
            pl.BlockSpec((N, D), lambda l, i: (0, 0)),
            pl.BlockSpec((64, D), lambda l, i: (0, 0)),
            pl.BlockSpec((64, D), lambda l, i: (0, 0)),
            pl.BlockSpec((D, 64), lambda l, i: (0, 0)),
            pl.BlockSpec((64, D), lambda l, i: (0, 0)),
            pl.BlockSpec((8, D), lambda l, i: (0, 0)),
        ],
        out_specs=pl.BlockSpec((8, D), lambda l, i: (0, 0)),
        out_shape=jax.ShapeDtypeStruct((8, D), jnp.float32),
        scratch_shapes=[
            pltpu.VMEM((N, D), jnp.bfloat16),
            pltpu.VMEM((1, D), jnp.float32),
            pltpu.VMEM((N, D), jnp.float32),
            pltpu.VMEM((1, 64), jnp.float32),
        ],
        compiler_params=pltpu.CompilerParams(
            dimension_semantics=("arbitrary", "arbitrary")),
    )(u, y2, w3, wd1, wd2, wd3, bias)
    return out[0, 0:1]


# speedup vs baseline: 3.0665x; 1.0033x over previous
import jax
import jax.numpy as jnp
from jax.experimental import pallas as pl
from jax.experimental.pallas import tpu as pltpu

N = 8192
BMA = 512
NBA = N // BMA
BMB = 1024
NBB = N // BMB
D = 128
HIGHEST = jax.lax.Precision.HIGHEST


def _pack_hi_lo(y):
    bits = jax.lax.bitcast_convert_type(y, jnp.uint32)
    hi = jax.lax.bitcast_convert_type(
        bits & jnp.uint32(0xFFFF0000), jnp.float32)
    lo = (y - hi).astype(jnp.bfloat16)
    return jnp.concatenate(
        [hi.astype(jnp.bfloat16)[:, :64], lo[:, :64]], axis=1)


def _init_kernel(xp_ref, w1_ref, y1p_ref, cs_ref):
    y1 = jnp.dot(xp_ref[...], w1_ref[...],
                 preferred_element_type=jnp.float32, precision=HIGHEST)
    y1p_ref[...] = _pack_hi_lo(y1)
    cs_ref[...] = 0.5 * jnp.sum(y1, axis=0, keepdims=True)


def _layer1_kernel(adj_ref, y1p_ref, cs_ref, w2_ref, b1_ref, u_ref, y2_ref):
    y1p = y1p_ref
    cs = cs_ref
    u = (adj_ref[...] - 0.5).astype(jnp.bfloat16)
    u_ref[...] = u
    c = jnp.dot(u, y1p[...], preferred_element_type=jnp.float32)
    h1 = jnp.maximum(
        c[:, :64] + c[:, 64:] + cs[0:1, :64] + b1_ref[0:1, :64], 0.0)
    y2_ref[...] = jnp.dot(h1, w2_ref[...],
                          preferred_element_type=jnp.float32,
                          precision=HIGHEST)


def _layer23_kernel(u_ref, y2_ref, w3_ref, wd1_ref, wd2_ref, wd3_ref, b_ref,
                    out_ref, ypk, cs, ynx, g):
    l = pl.program_id(0)
    i = pl.program_id(1)

    @pl.when(jnp.logical_and(l == 0, i == 0))
    def _start2():
        ypk[...] = _pack_hi_lo(y2_ref[...])
        cs[...] = 0.5 * jnp.sum(y2_ref[...], axis=0, keepdims=True)
        g[...] = jnp.zeros_like(g)

    @pl.when(jnp.logical_and(l == 1, i == 0))
    def _start3():
        ypk[...] = _pack_hi_lo(ynx[...])
        cs[...] = 0.5 * jnp.sum(ynx[...], axis=0, keepdims=True)

    c = jnp.dot(u_ref[...], ypk[...], preferred_element_type=jnp.float32)
    hsum = c[:, :64] + c[:, 64:] + cs[0:1, :64]

    @pl.when(l == 0)
    def _layer2():
        h2 = jnp.maximum(hsum + b_ref[1:2, :64], 0.0)
        ynx[pl.ds(i * BMB, BMB), :] = jnp.dot(
            h2, w3_ref[...], preferred_element_type=jnp.float32,
            precision=HIGHEST)

    @pl.when(l == 1)
    def _layer3():
        h3 = jnp.maximum(hsum + b_ref[2:3, :64], 0.0)
        g[...] += jnp.sum(h3, axis=0, keepdims=True)

    @pl.when(jnp.logical_and(l == 1, i == NBB - 1))
    def _head():
        gv = g[...]
        norm = jnp.maximum(jnp.sqrt(jnp.sum(gv * gv)), 1e-12)
        gn = gv / norm
        d1 = jnp.maximum(
            jnp.dot(gn, wd1_ref[...], preferred_element_type=jnp.float32,
                    precision=HIGHEST) + b_ref[3:4, :], 0.0)
        d2 = jnp.maximum(
            jnp.dot(d1, wd2_ref[...], preferred_element_type=jnp.float32,
                    precision=HIGHEST) + b_ref[4:5, :64], 0.0)
        d3 = (jnp.dot(d2, wd3_ref[...], preferred_element_type=jnp.float32,
                      precision=HIGHEST) + b_ref[5:6, :])
        out_ref[...] = jnp.broadcast_to(d3, out_ref.shape)


def _pad2(w, rows, cols):
    return jnp.pad(w, ((0, rows - w.shape[0]), (0, cols - w.shape[1])))


def kernel(x, adj, W1, b1, W2, b2, W3, b3, Wd1, bd1, Wd2, bd2, Wd3, bd3):
    xp = jnp.pad(x, ((0, 0), (0, D - x.shape[1])))
    w1 = _pad2(W1, D, D)
    w2 = _pad2(W2, 64, D)
    w3 = _pad2(W3, 64, D)
    wd1 = _pad2(Wd1, 64, D)
    wd2 = _pad2(Wd2, D, 64)
    wd3 = _pad2(Wd3, 64, D)
    bias = jnp.zeros((8, D), jnp.float32)
    bias = bias.at[0, :16].set(b1).at[1, :32].set(b2).at[2, :64].set(b3)
    bias = bias.at[3, :128].set(bd1).at[4, :64].set(bd2).at[5, :1].set(bd3)

    y1p, cs0 = pl.pallas_call(
        _init_kernel,
        grid=(1,),
        in_specs=[
            pl.BlockSpec((N, D), lambda i: (0, 0)),
            pl.BlockSpec((D, D), lambda i: (0, 0)),
        ],
        out_specs=[
            pl.BlockSpec((N, D), lambda i: (0, 0)),
            pl.BlockSpec((1, D), lambda i: (0, 0)),
        ],
        out_shape=[
            jax.ShapeDtypeStruct((N, D), jnp.bfloat16),
            jax.ShapeDtypeStruct((1, D), jnp.float32),
        ],
        compiler_params=pltpu.CompilerParams(
            dimension_semantics=("arbitrary",)),
    )(xp, w1)

    u, y2 = pl.pallas_call(
        _layer1_kernel,
        grid=(NBA,),
        in_specs=[
            pl.BlockSpec((BMA, N), lambda i: (i, 0)),
            pl.BlockSpec((N, D), lambda i: (0, 0)),
            pl.BlockSpec((1, D), lambda i: (0, 0)),
            pl.BlockSpec((64, D), lambda i: (0, 0)),
            pl.BlockSpec((8, D), lambda i: (0, 0)),
        ],
        out_specs=[
            pl.BlockSpec((BMA, N), lambda i: (i, 0)),
            pl.BlockSpec((BMA, D), lambda i: (i, 0)),
        ],
        out_shape=[
            jax.ShapeDtypeStruct((N, N), jnp.bfloat16),
            jax.ShapeDtypeStruct((N, D), jnp.float32),
        ],
        compiler_params=pltpu.CompilerParams(
            dimension_semantics=("arbitrary",)),
    )(adj, y1p, cs0, w2, bias)

    out = pl.pallas_call(
        _layer23_kernel,
        grid=(2, NBB),
        in_specs=[
            pl.BlockSpec((BMB, N), lambda l, i: (i, 0)),
            pl.BlockSpec((N, D), lambda l, i: (0, 0)),
            pl.BlockSpec((64, D), lambda l, i: (0, 0)),
            pl.BlockSpec((64, D), lambda l, i: (0, 0)),
            pl.BlockSpec((D, 64), lambda l, i: (0, 0)),
            pl.BlockSpec((64, D), lambda l, i: (0, 0)),
            pl.BlockSpec((8, D), lambda l, i: (0, 0)),
        ],
        out_specs=pl.BlockSpec((8, D), lambda l, i: (0, 0)),
        out_shape=jax.ShapeDtypeStruct((8, D), jnp.float32),
        scratch_shapes=[
            pltpu.VMEM((N, D), jnp.bfloat16),
            pltpu.VMEM((1, D), jnp.float32),
            pltpu.VMEM((N, D), jnp.float32),
            pltpu.VMEM((1, 64), jnp.float32),
        ],
        compiler_params=pltpu.CompilerParams(
            dimension_semantics=("arbitrary", "arbitrary")),
    )(u, y2, w3, wd1, wd2, wd3, bias)
    return out[0, 0:1]


# speedup vs baseline: 3.0670x; 1.0002x over previous
import jax
import jax.numpy as jnp
from jax.experimental import pallas as pl
from jax.experimental.pallas import tpu as pltpu

N = 8192
BMA = 512
NBA = N // BMA
BMB = 1024
NBB = N // BMB
D = 128
HIGHEST = jax.lax.Precision.HIGHEST


def _pack_hi_lo(y):
    bits = jax.lax.bitcast_convert_type(y, jnp.uint32)
    hi = jax.lax.bitcast_convert_type(
        bits & jnp.uint32(0xFFFF0000), jnp.float32)
    lo = (y - hi).astype(jnp.bfloat16)
    return jnp.concatenate(
        [hi.astype(jnp.bfloat16)[:, :64], lo[:, :64]], axis=1)


def _init_kernel(xp_ref, w1_ref, y1p_ref, cs_ref):
    y1 = jnp.dot(xp_ref[...], w1_ref[...],
                 preferred_element_type=jnp.float32, precision=HIGHEST)
    y1p_ref[...] = _pack_hi_lo(y1)
    cs_ref[...] = 0.5 * jnp.sum(y1, axis=0, keepdims=True)


def _layer1_kernel(adj_ref, y1p_ref, cs_ref, w2_ref, b1_ref, u_ref, y2_ref):
    y1p = y1p_ref
    cs = cs_ref
    u = (adj_ref[...] - 0.5).astype(jnp.bfloat16)
    u_ref[...] = u
    c = jnp.dot(u, y1p[...], preferred_element_type=jnp.float32)
    h1 = jnp.maximum(
        c[:, :64] + c[:, 64:] + cs[0:1, :64] + b1_ref[0:1, :64], 0.0)
    y2_ref[...] = jnp.dot(h1, w2_ref[...],
                          preferred_element_type=jnp.float32,
                          precision=HIGHEST)


def _layer23_kernel(u_ref, y2_ref, w3_ref, wd1_ref, wd2_ref, wd3_ref, b_ref,
                    out_ref, ypk, cs, ynx, g):
    l = pl.program_id(0)
    i = pl.program_id(1)

    @pl.when(jnp.logical_and(l == 0, i == 0))
    def _start2():
        ypk[...] = _pack_hi_lo(y2_ref[...])
        cs[...] = 0.5 * jnp.sum(y2_ref[...], axis=0, keepdims=True)
        g[...] = jnp.zeros_like(g)

    @pl.when(jnp.logical_and(l == 1, i == 0))
    def _start3():
        ypk[...] = _pack_hi_lo(ynx[...])
        cs[...] = 0.5 * jnp.sum(ynx[...], axis=0, keepdims=True)

    c = jnp.dot(u_ref[...], ypk[...], preferred_element_type=jnp.float32)
    hsum = c[:, :64] + c[:, 64:] + cs[0:1, :64]

    @pl.when(l == 0)
    def _layer2():
        h2 = jnp.maximum(hsum + b_ref[1:2, :64], 0.0)
        ynx[pl.ds(i * BMB, BMB), :] = jnp.dot(
            h2, w3_ref[...], preferred_element_type=jnp.float32,
            precision=HIGHEST)

    @pl.when(l == 1)
    def _layer3():
        h3 = jnp.maximum(hsum + b_ref[2:3, :64], 0.0)
        g[...] += jnp.sum(h3, axis=0, keepdims=True)

    @pl.when(jnp.logical_and(l == 1, i == NBB - 1))
    def _head():
        gv = g[...]
        norm = jnp.maximum(jnp.sqrt(jnp.sum(gv * gv)), 1e-12)
        gn = gv / norm
        d1 = jnp.maximum(
            jnp.dot(gn, wd1_ref[...], preferred_element_type=jnp.float32,
                    precision=HIGHEST) + b_ref[3:4, :], 0.0)
        d2 = jnp.maximum(
            jnp.dot(d1, wd2_ref[...], preferred_element_type=jnp.float32,
                    precision=HIGHEST) + b_ref[4:5, :64], 0.0)
        d3 = (jnp.dot(d2, wd3_ref[...], preferred_element_type=jnp.float32,
                      precision=HIGHEST) + b_ref[5:6, :])
        out_ref[...] = jnp.broadcast_to(d3, out_ref.shape)


def _pad2(w, rows, cols):
    return jnp.pad(w, ((0, rows - w.shape[0]), (0, cols - w.shape[1])))


def kernel(x, adj, W1, b1, W2, b2, W3, b3, Wd1, bd1, Wd2, bd2, Wd3, bd3):
    xp = jnp.pad(x, ((0, 0), (0, D - x.shape[1])))
    w1 = _pad2(W1, D, D)
    w2 = _pad2(W2, 64, D)
    w3 = _pad2(W3, 64, D)
    wd1 = _pad2(Wd1, 64, D)
    wd2 = _pad2(Wd2, D, 64)
    wd3 = _pad2(Wd3, 64, D)
    bias = jnp.zeros((8, D), jnp.float32)
    bias = bias.at[0, :16].set(b1).at[1, :32].set(b2).at[2, :64].set(b3)
    bias = bias.at[3, :128].set(bd1).at[4, :64].set(bd2).at[5, :1].set(bd3)

    y1p, cs0 = pl.pallas_call(
        _init_kernel,
        grid=(1,),
        in_specs=[
            pl.BlockSpec((N, D), lambda i: (0, 0)),
            pl.BlockSpec((D, D), lambda i: (0, 0)),
        ],
        out_specs=[
            pl.BlockSpec((N, D), lambda i: (0, 0)),
            pl.BlockSpec((1, D), lambda i: (0, 0)),
        ],
        out_shape=[
            jax.ShapeDtypeStruct((N, D), jnp.bfloat16),
            jax.ShapeDtypeStruct((1, D), jnp.float32),
        ],
        compiler_params=pltpu.CompilerParams(
            dimension_semantics=("arbitrary",)),
    )(xp, w1)

    u, y2 = pl.pallas_call(
        _layer1_kernel,
        grid=(NBA,),
        in_specs=[
            pl.BlockSpec((BMA, N), lambda i: (i, 0)),
            pl.BlockSpec((N, D), lambda i: (0, 0)),
            pl.BlockSpec((1, D), lambda i: (0, 0)),
            pl.BlockSpec((64, D), lambda i: (0, 0)),
            pl.BlockSpec((8, D), lambda i: (0, 0)),
        ],
        out_specs=[
            pl.BlockSpec((BMA, N), lambda i: (i, 0)),
            pl.BlockSpec((BMA, D), lambda i: (i, 0)),
        ],
        out_shape=[
            jax.ShapeDtypeStruct((N, N), jnp.bfloat16),
            jax.ShapeDtypeStruct((N, D), jnp.float32),
        ],
        compiler_params=pltpu.CompilerParams(
            dimension_semantics=("parallel",)),
    )(adj, y1p, cs0, w2, bias)

    out = pl.pallas_call(
        _layer23_kernel,
        grid=(2, NBB),
        in_specs=[
            pl.BlockSpec((BMB, N), lambda l, i: (i, 0)),
            pl.BlockSpec((N, D), lambda l, i: (0, 0)),
            pl.BlockSpec((64, D), lambda l, i: (0, 0)),
            pl.BlockSpec((64, D), lambda l, i: (0, 0)),
            pl.BlockSpec((D, 64), lambda l, i: (0, 0)),
            pl.BlockSpec((64, D), lambda l, i: (0, 0)),
            pl.BlockSpec((8, D), lambda l, i: (0, 0)),
        ],
        out_specs=pl.BlockSpec((8, D), lambda l, i: (0, 0)),
        out_shape=jax.ShapeDtypeStruct((8, D), jnp.float32),
        scratch_shapes=[
            pltpu.VMEM((N, D), jnp.bfloat16),
            pltpu.VMEM((1, D), jnp.float32),
            pltpu.VMEM((N, D), jnp.float32),
            pltpu.VMEM((1, 64), jnp.float32),
        ],
        compiler_params=pltpu.CompilerParams(
            dimension_semantics=("arbitrary", "arbitrary")),
    )(u, y2, w3, wd1, wd2, wd3, bias)
    return out[0, 0:1]


# speedup vs baseline: 3.0706x; 1.0012x over previous
import jax
import jax.numpy as jnp
from jax.experimental import pallas as pl
from jax.experimental.pallas import tpu as pltpu

N = 8192
BMA = 512
NBA = N // BMA
BMB = 1024
NBB = N // BMB
D = 128
HIGHEST = jax.lax.Precision.HIGHEST


def _pack_hi_lo(y):
    bits = jax.lax.bitcast_convert_type(y, jnp.uint32)
    hi = jax.lax.bitcast_convert_type(
        bits & jnp.uint32(0xFFFF0000), jnp.float32)
    lo = (y - hi).astype(jnp.bfloat16)
    return jnp.concatenate(
        [hi.astype(jnp.bfloat16)[:, :64], lo[:, :64]], axis=1)


def _init_kernel(xp_ref, w1_ref, y1p_ref, cs_ref):
    y1 = jnp.dot(xp_ref[...], w1_ref[...],
                 preferred_element_type=jnp.float32, precision=HIGHEST)
    y1p_ref[...] = _pack_hi_lo(y1)
    cs_ref[...] = jnp.broadcast_to(
        0.5 * jnp.sum(y1, axis=0, keepdims=True), cs_ref.shape)


def _layer1_kernel(adj_ref, y1p_ref, cs_ref, w2_ref, b1_ref,
                   u_ref, y2p_ref, cs2_ref, cs2s):
    i = pl.program_id(0)

    @pl.when(i == 0)
    def _zero():
        cs2s[...] = jnp.zeros_like(cs2s)

    u = (adj_ref[...] - 0.5).astype(jnp.bfloat16)
    u_ref[...] = u
    c = jnp.dot(u, y1p_ref[...], preferred_element_type=jnp.float32)
    h1 = jnp.maximum(
        c[:, :64] + c[:, 64:] + cs_ref[0:1, :64] + b1_ref[0:1, :64], 0.0)
    y2 = jnp.dot(h1, w2_ref[...], preferred_element_type=jnp.float32,
                 precision=HIGHEST)
    y2p_ref[...] = _pack_hi_lo(y2)
    cs2s[...] += 0.5 * jnp.sum(y2, axis=0, keepdims=True)

    @pl.when(i == NBA - 1)
    def _emit():
        cs2_ref[...] = jnp.broadcast_to(cs2s[...], cs2_ref.shape)


def _layer23_kernel(u_ref, y2p_ref, cs2_ref, w3_ref, wd1_ref, wd2_ref,
                    wd3_ref, b_ref, out_ref, ypk3, cs3, g):
    l = pl.program_id(0)
    i = pl.program_id(1)

    @pl.when(jnp.logical_and(l == 0, i == 0))
    def _zero():
        cs3[...] = jnp.zeros_like(cs3)
        g[...] = jnp.zeros_like(g)

    @pl.when(l == 0)
    def _layer2():
        c = jnp.dot(u_ref[...], y2p_ref[...],
                    preferred_element_type=jnp.float32)
        h2 = jnp.maximum(
            c[:, :64] + c[:, 64:] + cs2_ref[0:1, :64] + b_ref[1:2, :64], 0.0)
        y3 = jnp.dot(h2, w3_ref[...], preferred_element_type=jnp.float32,
                     precision=HIGHEST)
        ypk3[pl.ds(i * BMB, BMB), :] = _pack_hi_lo(y3)
        cs3[...] += 0.5 * jnp.sum(y3, axis=0, keepdims=True)

    @pl.when(l == 1)
    def _layer3():
        c = jnp.dot(u_ref[...], ypk3[...],
                    preferred_element_type=jnp.float32)
        h3 = jnp.maximum(
            c[:, :64] + c[:, 64:] + cs3[0:1, :64] + b_ref[2:3, :64], 0.0)
        g[...] += jnp.sum(h3, axis=0, keepdims=True)

    @pl.when(jnp.logical_and(l == 1, i == NBB - 1))
    def _head():
        gv = g[...]
        norm = jnp.maximum(jnp.sqrt(jnp.sum(gv * gv)), 1e-12)
        gn = gv / norm
        d1 = jnp.maximum(
            jnp.dot(gn, wd1_ref[...], preferred_element_type=jnp.float32,
                    precision=HIGHEST) + b_ref[3:4, :], 0.0)
        d2 = jnp.maximum(
            jnp.dot(d1, wd2_ref[...], preferred_element_type=jnp.float32,
                    precision=HIGHEST) + b_ref[4:5, :64], 0.0)
        d3 = (jnp.dot(d2, wd3_ref[...], preferred_element_type=jnp.float32,
                      precision=HIGHEST) + b_ref[5:6, :])
        out_ref[...] = jnp.broadcast_to(d3, out_ref.shape)


def _pad2(w, rows, cols):
    return jnp.pad(w, ((0, rows - w.shape[0]), (0, cols - w.shape[1])))


def kernel(x, adj, W1, b1, W2, b2, W3, b3, Wd1, bd1, Wd2, bd2, Wd3, bd3):
    xp = jnp.pad(x, ((0, 0), (0, D - x.shape[1])))
    w1 = _pad2(W1, D, D)
    w2 = _pad2(W2, 64, D)
    w3 = _pad2(W3, 64, D)
    wd1 = _pad2(Wd1, 64, D)
    wd2 = _pad2(Wd2, D, 64)
    wd3 = _pad2(Wd3, 64, D)
    bias = jnp.zeros((8, D), jnp.float32)
    bias = bias.at[0, :16].set(b1).at[1, :32].set(b2).at[2, :64].set(b3)
    bias = bias.at[3, :128].set(bd1).at[4, :64].set(bd2).at[5, :1].set(bd3)

    y1p, cs1 = pl.pallas_call(
        _init_kernel,
        grid=(1,),
        in_specs=[
            pl.BlockSpec((N, D), lambda i: (0, 0)),
            pl.BlockSpec((D, D), lambda i: (0, 0)),
        ],
        out_specs=[
            pl.BlockSpec((N, D), lambda i: (0, 0)),
            pl.BlockSpec((8, D), lambda i: (0, 0)),
        ],
        out_shape=[
            jax.ShapeDtypeStruct((N, D), jnp.bfloat16),
            jax.ShapeDtypeStruct((8, D), jnp.float32),
        ],
        compiler_params=pltpu.CompilerParams(
            dimension_semantics=("arbitrary",)),
    )(xp, w1)

    u, y2p, cs2 = pl.pallas_call(
        _layer1_kernel,
        grid=(NBA,),
        in_specs=[
            pl.BlockSpec((BMA, N), lambda i: (i, 0)),
            pl.BlockSpec((N, D), lambda i: (0, 0)),
            pl.BlockSpec((8, D), lambda i: (0, 0)),
            pl.BlockSpec((64, D), lambda i: (0, 0)),
            pl.BlockSpec((8, D), lambda i: (0, 0)),
        ],
        out_specs=[
            pl.BlockSpec((BMA, N), lambda i: (i, 0)),
            pl.BlockSpec((BMA, D), lambda i: (i, 0)),
            pl.BlockSpec((8, D), lambda i: (0, 0)),
        ],
        out_shape=[
            jax.ShapeDtypeStruct((N, N), jnp.bfloat16),
            jax.ShapeDtypeStruct((N, D), jnp.bfloat16),
            jax.ShapeDtypeStruct((8, D), jnp.float32),
        ],
        scratch_shapes=[
            pltpu.VMEM((1, D), jnp.float32),
        ],
        compiler_params=pltpu.CompilerParams(
            dimension_semantics=("arbitrary",)),
    )(adj, y1p, cs1, w2, bias)

    out = pl.pallas_call(
        _layer23_kernel,
        grid=(2, NBB),
        in_specs=[
            pl.BlockSpec((BMB, N), lambda l, i: (i, 0)),
            pl.BlockSpec((N, D), lambda l, i: (0, 0)),
            pl.BlockSpec((8, D), lambda l, i: (0, 0)),
            pl.BlockSpec((64, D), lambda l, i: (0, 0)),
            pl.BlockSpec((64, D), lambda l, i: (0, 0)),
            pl.BlockSpec((D, 64), lambda l, i: (0, 0)),
            pl.BlockSpec((64, D), lambda l, i: (0, 0)),
            pl.BlockSpec((8, D), lambda l, i: (0, 0)),
        ],
        out_specs=pl.BlockSpec((8, D), lambda l, i: (0, 0)),
        out_shape=jax.ShapeDtypeStruct((8, D), jnp.float32),
        scratch_shapes=[
            pltpu.VMEM((N, D), jnp.bfloat16),
            pltpu.VMEM((1, D), jnp.float32),
            pltpu.VMEM((1, 64), jnp.float32),
        ],
        compiler_params=pltpu.CompilerParams(
            dimension_semantics=("arbitrary", "arbitrary")),
    )(u, y2p, cs2, w3, wd1, wd2, wd3, bias)
    return out[0, 0:1]
